# Initial kernel scaffold; baseline (speedup 1.0000x reference)
import jax
import jax.numpy as jnp
from jax.experimental import pallas as pl


def kernel(f_in, pos, w0, w1, w2, w3):
    raise NotImplementedError("write your pallas kernel here")


# speedup vs baseline: 1.4343x; 1.4343x over previous
import math
import functools

import jax
import jax.numpy as jnp
import numpy as np
from jax.experimental import pallas as pl
from jax.experimental.pallas import tpu as pltpu

MAX_RADIUS = 8.0
NUM_BASIS = 4
BATCH = 4
N = 512
D_IN = 16
MUL = 48
_SILU_CST = 1.679177

_fac = math.factorial


def _wigner3j_scalar(l1, l2, l3, m1, m2, m3):
    if m1 + m2 + m3 != 0:
        return 0.0
    pref = (-1.0) ** (l1 - l2 - m3) * math.sqrt(
        _fac(l1 + l2 - l3) * _fac(l1 - l2 + l3) * _fac(-l1 + l2 + l3)
        / _fac(l1 + l2 + l3 + 1)
        * _fac(l1 + m1) * _fac(l1 - m1) * _fac(l2 + m2) * _fac(l2 - m2)
        * _fac(l3 + m3) * _fac(l3 - m3))
    s = 0.0
    for t in range(l1 + l2 + l3 + 1):
        ds = [t, l3 - l2 + t + m1, l3 - l1 + t - m2,
              l1 + l2 - l3 - t, l1 - t - m1, l2 - t + m2]
        if any(d < 0 for d in ds):
            continue
        den = 1
        for q in ds:
            den *= _fac(q)
        s += (-1.0) ** t / den
    return pref * s


def _real_basis(l):
    U = np.zeros((2 * l + 1, 2 * l + 1), dtype=np.complex128)
    for m in range(-l, l + 1):
        if m > 0:
            U[l + m, l - m] = 1.0 / math.sqrt(2.0)
            U[l + m, l + m] = (-1.0) ** m / math.sqrt(2.0)
        elif m == 0:
            U[l, l] = 1.0
        else:
            U[l + m, l + m] = 1j / math.sqrt(2.0)
            U[l + m, l - m] = -1j * (-1.0) ** (-m) / math.sqrt(2.0)
    return U


def _w3j_real_np(l1, l2, l3):
    C = np.zeros((2 * l1 + 1, 2 * l2 + 1, 2 * l3 + 1), dtype=np.complex128)
    for m1 in range(-l1, l1 + 1):
        for m2 in range(-l2, l2 + 1):
            m3 = -m1 - m2
            if -l3 <= m3 <= l3:
                C[l1 + m1, l2 + m2, l3 + m3] = _wigner3j_scalar(l1, l2, l3, m1, m2, m3)
    Cr = np.einsum('ai,bj,ck,ijk->abc', _real_basis(l1), _real_basis(l2), _real_basis(l3), C)
    re, im = np.real(Cr), np.imag(Cr)
    out = re if np.linalg.norm(re) >= np.linalg.norm(im) else im
    return out.astype(np.float32)


_PATHS = [(0, 0, 0), (1, 1, 0), (2, 2, 0),
          (0, 1, 1), (1, 0, 1), (1, 2, 1), (2, 1, 1), (3, 2, 1),
          (0, 2, 2), (1, 1, 2), (2, 0, 2), (2, 2, 2), (3, 1, 2)]
_IN_SLICE = {0: (0, 1), 1: (1, 4), 2: (4, 9), 3: (9, 16)}
_SH_SLICE = {0: (0, 1), 1: (1, 4), 2: (4, 9)}
_NPATH_OUT = {0: 3, 1: 5, 2: 5}
NPATH = len(_PATHS)
C_OUT = MUL * 9

_CDIM = sum(2 * l3 + 1 for (_, _, l3) in _PATHS)


def _build_cmat():
    M = np.zeros((D_IN * 9, _CDIM), dtype=np.float32)
    off = 0
    offsets = []
    for (l1, l2, l3) in _PATHS:
        w3j = _w3j_real_np(l1, l2, l3)
        a0, a1 = _IN_SLICE[l1]
        s0, s1 = _SH_SLICE[l2]
        dk = 2 * l3 + 1
        alpha = math.sqrt((2 * l3 + 1) / _NPATH_OUT[l3])
        for a in range(a0, a1):
            for s in range(s0, s1):
                M[a * 9 + s, off:off + dk] = alpha * w3j[a - a0, s - s0, :]
        offsets.append(off)
        off += dk
    return M, offsets


_CMAT_NP, _COL_OFF = _build_cmat()

TJ = 128
TI = 16
E_TILE = TI * TJ
N_I = N // TI
C_PAD = 440


def _edge_kernel(pos_ref, f_ref, w0_ref, w1_ref, w2_ref, w3_ref, cmat_ref, out_ref):
    jb = pl.program_id(1)
    pos_j = pos_ref[0, pl.ds(jb * TJ, TJ), :]
    w0 = w0_ref[...]
    w1 = w1_ref[...]
    w2 = w2_ref[...]
    w3 = w3_ref[...]
    cmat = cmat_ref[...]

    jx = pos_j[:, 0][None, :]
    jy = pos_j[:, 1][None, :]
    jz = pos_j[:, 2][None, :]
    j_idx = jb * TJ + jax.lax.broadcasted_iota(jnp.int32, (TI, TJ), 1)

    step = MAX_RADIUS / (NUM_BASIS + 1)
    s3 = math.sqrt(3.0)
    lk_pairs = [(l3, k) for l3 in range(3) for k in range(2 * l3 + 1)]

    def body(ib, carry):
        accs, cnt = carry
        pos_i = pos_ref[0, pl.ds(ib * TI, TI), :]
        f_i = f_ref[0, pl.ds(ib * TI, TI), :]
        dx = jx - pos_i[:, 0][:, None]
        dy = jy - pos_i[:, 1][:, None]
        dz = jz - pos_i[:, 2][:, None]
        r2 = dx * dx + dy * dy + dz * dz
        i_idx = ib * TI + jax.lax.broadcasted_iota(jnp.int32, (TI, TJ), 0)
        mask = (r2 < MAX_RADIUS * MAX_RADIUS) & (i_idx != j_idx)
        r = jnp.sqrt(r2)
        inv = jnp.where(r > 0, 1.0 / jnp.where(r > 0, r, 1.0), 0.0)
        ux, uy, uz = dx * inv, dy * inv, dz * inv

        sh = jnp.stack([
            jnp.ones_like(ux), uy, uz, ux,
            s3 * ux * uy, s3 * uy * uz, 0.5 * (3.0 * uz * uz - 1.0),
            s3 * uz * ux, 0.5 * s3 * (ux * ux - uy * uy),
        ], axis=-1).reshape(E_TILE, 9)

        emb = jnp.stack([
            jnp.exp(-((r - step * (nb + 1)) / step) ** 2) * (2.0 / 1.12)
            for nb in range(NUM_BASIS)
        ], axis=-1).reshape(E_TILE, NUM_BASIS)

        h = jax.nn.silu(jnp.dot(emb, w0, preferred_element_type=jnp.float32)) * _SILU_CST
        h = jax.nn.silu(jnp.dot(h, w1, preferred_element_type=jnp.float32)) * _SILU_CST
        h = jax.nn.silu(jnp.dot(h, w2, preferred_element_type=jnp.float32)) * _SILU_CST
        wts = jnp.dot(h, w3, preferred_element_type=jnp.float32)
        maskf = mask.astype(jnp.float32)
        wm = wts * maskf.reshape(E_TILE, 1)

        f_e = jnp.broadcast_to(f_i[:, None, :], (TI, TJ, D_IN)).reshape(E_TILE, D_IN)
        feat = jnp.concatenate([f_e[:, a:a + 1] * sh for a in range(D_IN)], axis=1)
        call = jnp.dot(feat, cmat, preferred_element_type=jnp.float32)

        new_accs = []
        for (l3, k) in lk_pairs:
            blk = jnp.zeros((E_TILE, MUL), dtype=jnp.float32)
            for pi, (l1, l2, pl3) in enumerate(_PATHS):
                if pl3 != l3:
                    continue
                col = call[:, _COL_OFF[pi] + k:_COL_OFF[pi] + k + 1]
                blk = blk + wm[:, pi * MUL:(pi + 1) * MUL] * col
            new_accs.append(accs[len(new_accs)]
                            + blk.reshape(TI, TJ, MUL).sum(axis=0))
        cnt = cnt + maskf.sum(axis=0)[None, :]
        return tuple(new_accs), cnt

    accs0 = tuple(jnp.zeros((TJ, MUL), dtype=jnp.float32) for _ in lk_pairs)
    cnt0 = jnp.zeros((1, TJ), dtype=jnp.float32)
    accs, cnt = jax.lax.fori_loop(0, N_I, body, (accs0, cnt0))

    pieces = []
    idx = 0
    for l3 in range(3):
        dk = 2 * l3 + 1
        grp = jnp.stack(accs[idx:idx + dk], axis=-1)
        pieces.append(grp.reshape(TJ, MUL * dk))
        idx += dk
    out = jnp.concatenate(
        pieces + [cnt.reshape(TJ, 1),
                  jnp.zeros((TJ, C_PAD - C_OUT - 1), jnp.float32)],
        axis=-1)
    out_ref[0] = out


@jax.jit
def kernel(f_in, pos, w0, w1, w2, w3):
    w0n = w0 / math.sqrt(w0.shape[0])
    w1n = w1 / math.sqrt(w1.shape[0])
    w2n = w2 / math.sqrt(w2.shape[0])
    w3n = w3 / math.sqrt(w3.shape[0])
    cmat = jnp.asarray(_CMAT_NP)

    grid = (BATCH, N // TJ)
    raw = pl.pallas_call(
        _edge_kernel,
        grid=grid,
        in_specs=[
            pl.BlockSpec((1, N, 3), lambda b, j: (b, 0, 0)),
            pl.BlockSpec((1, N, D_IN), lambda b, j: (b, 0, 0)),
            pl.BlockSpec(w0.shape, lambda b, j: (0, 0)),
            pl.BlockSpec(w1.shape, lambda b, j: (0, 0)),
            pl.BlockSpec(w2.shape, lambda b, j: (0, 0)),
            pl.BlockSpec(w3.shape, lambda b, j: (0, 0)),
            pl.BlockSpec(_CMAT_NP.shape, lambda b, j: (0, 0)),
        ],
        out_specs=pl.BlockSpec((1, TJ, C_PAD), lambda b, j: (b, j, 0)),
        out_shape=jax.ShapeDtypeStruct((BATCH, N, C_PAD), jnp.float32),
        compiler_params=pltpu.CompilerParams(
            dimension_semantics=("parallel", "parallel")),
    )(pos, f_in, w0n, w1n, w2n, w3n, cmat)

    num_edges = jnp.sum(raw[:, :, C_OUT])
    scale = jax.lax.rsqrt(num_edges / (BATCH * N))
    return raw[:, :, :C_OUT] * scale
